# baseline (speedup 1.0000x reference)
import functools

import jax
import jax.numpy as jnp
from jax import lax
from jax.experimental import pallas as pl
from jax.experimental.pallas import tpu as pltpu
from jax.experimental.pallas import tpu_sc as plsc

_B = 1024
_NI = 512
_NO = 64
_H = 128

_OB = 16
_G = _NO // _OB

_NC = 2
_NS = 16
_NW = _NC * _NS
_BPW = _B // _NW
_L = 16


def _scores_body(l1_ref, l2_ref, l3_ref, state_ref, scores_ref, v_s):
    o = pl.program_id(0)

    @pl.when(o < _G)
    def _build_v():
        l1b = l1_ref[...]
        l2b = l2_ref[...]
        l3b = l3_ref[...]
        u = lax.dot_general(l3b, l2b, (((1,), (2,)), ((0,), (0,))),
                            preferred_element_type=jnp.float32)
        vrow = lax.dot_general(u, l1b, (((2,), (2,)), ((0,), (0,))),
                               preferred_element_type=jnp.float32)
        v_s[pl.ds(o * _OB, _OB), :] = vrow.reshape(_OB, _NI)

    @pl.when(o == _G)
    def _contract():
        scores_ref[...] = lax.dot_general(
            state_ref[...], v_s[...], (((1,), (1,)), ((), ())),
            preferred_element_type=jnp.float32)


def _tc_scores(linear1, linear2, linear3, state):
    clamp = lambda o: (jnp.minimum(o, _G - 1), 0, 0)
    return pl.pallas_call(
        _scores_body,
        grid=(_G + 1,),
        in_specs=[
            pl.BlockSpec((_OB, _NI, _H), clamp),
            pl.BlockSpec((_OB, _H, _H), clamp),
            pl.BlockSpec((_OB, _H, 1), clamp),
            pl.BlockSpec((_B, _NI), lambda o: (0, 0)),
        ],
        out_specs=pl.BlockSpec((_B, _NO), lambda o: (0, 0)),
        out_shape=jax.ShapeDtypeStruct((_B, _NO), jnp.float32),
        scratch_shapes=[pltpu.VMEM((_NO, _NI), jnp.float32)],
    )(linear1, linear2, linear3, state)


@functools.partial(
    pl.kernel,
    mesh=plsc.VectorSubcoreMesh(core_axis_name="c", subcore_axis_name="s"),
    out_type=jax.ShapeDtypeStruct((_B,), jnp.float32),
    scratch_types=[
        pltpu.VMEM((_BPW,), jnp.int32),
        pltpu.VMEM((_BPW * _NO,), jnp.float32),
        pltpu.VMEM((_BPW,), jnp.float32),
    ],
    compiler_params=pltpu.CompilerParams(needs_layout_passes=False),
)
def _sc_select(scores_hbm, opt_hbm, out_hbm, idx_v, sc_v, out_v):
    wid = lax.axis_index("s") * _NC + lax.axis_index("c")
    base = wid * _BPW
    pltpu.sync_copy(scores_hbm.at[pl.ds(base * _NO, _BPW * _NO)], sc_v)
    pltpu.sync_copy(opt_hbm.at[pl.ds(base, _BPW)], idx_v)
    for g in range(_BPW // _L):
        tok = g * _L + lax.broadcasted_iota(jnp.int32, (_L,), 0)
        fidx = tok * _NO + idx_v[pl.ds(g * _L, _L)]
        out_v[pl.ds(g * _L, _L)] = plsc.load_gather(sc_v, [fidx])
    pltpu.sync_copy(out_v, out_hbm.at[pl.ds(base, _BPW)])


def kernel(state, option, action, linear1, linear2, linear3):
    scores = _tc_scores(linear1, linear2, linear3, state)
    opt = option.astype(jnp.int32).reshape(_B)
    out = _sc_select(scores.reshape(_B * _NO), opt)
    return out.reshape(_B, 1)


# speedup vs baseline: 1.9414x; 1.9414x over previous
import jax
import jax.numpy as jnp
from jax import lax
from jax.experimental import pallas as pl
from jax.experimental.pallas import tpu as pltpu

_B = 1024
_NI = 512
_NO = 64
_H = 128

_NCH = 8
_OC = _NO // _NCH


def _body(l1_hbm, l2_hbm, l3_hbm, state_hbm, opt_hbm, out_ref,
          l1_v, l2_v, l3_v, state_v, opt_v, v_s, sems):
    cp_l2 = pltpu.make_async_copy(l2_hbm, l2_v, sems.at[_NCH])
    cp_l3 = pltpu.make_async_copy(l3_hbm, l3_v, sems.at[_NCH + 1])
    cp_st = pltpu.make_async_copy(state_hbm, state_v, sems.at[_NCH + 2])
    cp_opt = pltpu.make_async_copy(opt_hbm, opt_v, sems.at[_NCH + 3])
    cp_l2.start()
    cp_l3.start()
    cp_st.start()
    cp_opt.start()
    cps = []
    for k in range(_NCH):
        cp = pltpu.make_async_copy(l1_hbm.at[pl.ds(k * _OC, _OC)],
                                   l1_v.at[pl.ds(k * _OC, _OC)],
                                   sems.at[k])
        cp.start()
        cps.append(cp)

    cp_l2.wait()
    cp_l3.wait()
    u = lax.dot_general(l3_v[...], l2_v[...], (((1,), (2,)), ((0,), (0,))),
                        preferred_element_type=jnp.float32)

    for k in range(_NCH):
        cps[k].wait()
        l1b = l1_v[pl.ds(k * _OC, _OC)]
        uk = u[k * _OC:(k + 1) * _OC]
        vrow = lax.dot_general(uk, l1b, (((2,), (2,)), ((0,), (0,))),
                               preferred_element_type=jnp.float32)
        v_s[pl.ds(k * _OC, _OC), :] = vrow.reshape(_OC, _NI)

    cp_st.wait()
    cp_opt.wait()
    scores = lax.dot_general(state_v[...], v_s[...], (((1,), (1,)), ((), ())),
                             preferred_element_type=jnp.float32)
    onehot = (opt_v[...] == lax.broadcasted_iota(jnp.int32, (1, _NO), 1))
    out_ref[...] = jnp.sum(jnp.where(onehot, scores, 0.0), axis=1,
                           keepdims=True)


def kernel(state, option, action, linear1, linear2, linear3):
    opt = option.astype(jnp.int32).reshape(_B, 1)
    hbm = pl.BlockSpec(memory_space=pltpu.MemorySpace.HBM)
    out = pl.pallas_call(
        _body,
        in_specs=[hbm, hbm, hbm, hbm, hbm],
        out_specs=pl.BlockSpec(memory_space=pltpu.MemorySpace.VMEM),
        out_shape=jax.ShapeDtypeStruct((_B, 1), jnp.float32),
        scratch_shapes=[
            pltpu.VMEM((_NO, _NI, _H), jnp.float32),
            pltpu.VMEM((_NO, _H, _H), jnp.float32),
            pltpu.VMEM((_NO, _H, 1), jnp.float32),
            pltpu.VMEM((_B, _NI), jnp.float32),
            pltpu.VMEM((_B, 1), jnp.int32),
            pltpu.VMEM((_NO, _NI), jnp.float32),
            pltpu.SemaphoreType.DMA((_NCH + 4,)),
        ],
    )(linear1, linear2, linear3, state, opt)
    return out
